# baseline (speedup 1.0000x reference)
import jax
import jax.numpy as jnp
from jax import lax
from jax.experimental import pallas as pl
from jax.experimental.pallas import tpu as pltpu
from jax.experimental.pallas import tpu_sc as plsc

NC = 2
NS = 16
NW = NC * NS

BATCH = 16384
EMBED = 64
BPW = BATCH // NW
G = 16
NG = BPW // G
LANES = 16
KV = EMBED // LANES


def _make_kernel():
  mesh = plsc.VectorSubcoreMesh(core_axis_name="c", subcore_axis_name="s")

  @pl.kernel(
      out_type=jax.ShapeDtypeStruct((BATCH, 2 * EMBED), jnp.float32),
      mesh=mesh,
      scratch_types=[
          pltpu.VMEM((2, BPW), jnp.int32),
          pltpu.VMEM((2, 2 * G, 8, EMBED), jnp.float32),
          pltpu.VMEM((2, G, 2 * EMBED), jnp.float32),
          pltpu.SemaphoreType.DMA,
          pltpu.SemaphoreType.DMA,
      ],
  )
  def k(idx_hbm, ctab_hbm, qtab_hbm, out_hbm, idx_s, faces, mix, gsem, osem):
    wid = lax.axis_index("s") * NC + lax.axis_index("c")
    base = wid * BPW

    pltpu.sync_copy(idx_hbm.at[wid], idx_s)

    def issue(g, slot):
      cvec = idx_s[0, pl.ds(g * G, G)]
      qvec = idx_s[1, pl.ds(g * G, G)]
      for j in range(G):
        pltpu.async_copy(ctab_hbm.at[cvec[j] >> 3], faces.at[slot, j], gsem)
        pltpu.async_copy(qtab_hbm.at[qvec[j] >> 3], faces.at[slot, G + j], gsem)

    issue(0, 0)

    def body(g, _):
      slot = g % 2

      @pl.when(g + 1 < NG)
      def _():
        issue(g + 1, (g + 1) % 2)

      for j in range(2 * G):
        pltpu.make_async_copy(ctab_hbm.at[0], faces.at[slot, j], gsem).wait()

      cvec = idx_s[0, pl.ds(g * G, G)]
      qvec = idx_s[1, pl.ds(g * G, G)]
      for j in range(G):
        cr = cvec[j] & 7
        qr = qvec[j] & 7
        for t in range(KV):
          mix[slot, j, pl.ds(t * LANES, LANES)] = faces[
              slot, j, cr, pl.ds(t * LANES, LANES)
          ]
          mix[slot, j, pl.ds(EMBED + t * LANES, LANES)] = faces[
              slot, G + j, qr, pl.ds(t * LANES, LANES)
          ]

      @pl.when(g >= 2)
      def _():
        pltpu.make_async_copy(
            mix.at[slot], out_hbm.at[pl.ds(base, G)], osem
        ).wait()

      pltpu.async_copy(mix.at[slot], out_hbm.at[pl.ds(base + g * G, G)], osem)
      return 0

    lax.fori_loop(0, NG, body, 0)
    for _ in range(2):
      pltpu.make_async_copy(mix.at[0], out_hbm.at[pl.ds(base, G)], osem).wait()

  return k


_kernel = _make_kernel()


@jax.jit
def kernel(x, color_table, question_table):
  xi = x.astype(jnp.int32).T.reshape(2, NW, BPW).transpose(1, 0, 2)
  ctab3 = color_table.reshape(-1, 8, EMBED)
  qtab3 = question_table.reshape(-1, 8, EMBED)
  return _kernel(xi, ctab3, qtab3)
